# baseline (speedup 1.0000x reference)
import jax
import jax.numpy as jnp
from jax.experimental import pallas as pl
from jax.experimental.pallas import tpu as pltpu


def _copy_kernel(u_in, i_in, u_out, i_out, sem_u, sem_i):
    cu = pltpu.make_async_copy(u_in, u_out, sem_u)
    ci = pltpu.make_async_copy(i_in, i_out, sem_i)
    cu.start()
    ci.start()
    cu.wait()
    ci.wait()


def kernel(user_weight, item_weight):
    return pl.pallas_call(
        _copy_kernel,
        out_shape=(
            jax.ShapeDtypeStruct(user_weight.shape, user_weight.dtype),
            jax.ShapeDtypeStruct(item_weight.shape, item_weight.dtype),
        ),
        in_specs=[
            pl.BlockSpec(memory_space=pltpu.MemorySpace.HBM),
            pl.BlockSpec(memory_space=pltpu.MemorySpace.HBM),
        ],
        out_specs=(
            pl.BlockSpec(memory_space=pltpu.MemorySpace.HBM),
            pl.BlockSpec(memory_space=pltpu.MemorySpace.HBM),
        ),
        scratch_shapes=[pltpu.SemaphoreType.DMA, pltpu.SemaphoreType.DMA],
    )(user_weight, item_weight)


# speedup vs baseline: 15.6525x; 15.6525x over previous
import jax
import jax.numpy as jnp
from jax.experimental import pallas as pl
from jax.experimental.pallas import tpu as pltpu

_ROWS = 100000
_BLK = 10000


def _copy_kernel(u_in, i_in, u_out, i_out):
    u_out[...] = u_in[...]
    i_out[...] = i_in[...]


def kernel(user_weight, item_weight):
    grid = _ROWS // _BLK
    spec = pl.BlockSpec((_BLK, 64), lambda n: (n, 0))
    return pl.pallas_call(
        _copy_kernel,
        grid=(grid,),
        out_shape=(
            jax.ShapeDtypeStruct(user_weight.shape, user_weight.dtype),
            jax.ShapeDtypeStruct(item_weight.shape, item_weight.dtype),
        ),
        in_specs=[spec, spec],
        out_specs=(spec, spec),
    )(user_weight, item_weight)
